# baseline (speedup 1.0000x reference)
import functools

import jax
import jax.numpy as jnp
from jax import lax
from jax.experimental import pallas as pl
from jax.experimental.pallas import tpu as pltpu
from jax.experimental.pallas import tpu_sc as plsc

_NC = 2
_NS = 16
_NW = _NC * _NS
_L = 16
_BT = 128
_TPW = 4
_CHUNK = _TPW * _BT
_NBUF = 2


def _emb_body(table_hbm, idx_hbm, out_hbm, idx_v, rows_v, skew_v, idxtab,
              trans_v, sems, *, nb, t_len, d):
    wid = lax.axis_index("s") * _NC + lax.axis_index("c")
    ibase = wid * _CHUNK
    obase = wid * _TPW * _BT * 8
    nct = d // 8
    sem_idx, sem_gth, sem_out = sems
    lane = lax.iota(jnp.int32, _L)
    for r in range(_L):
        idxtab[r] = jnp.bitwise_and(lane + r, _L - 1) + lane * _L
    for c in range(_L):
        idxtab[_L + c] = jnp.bitwise_and(lane + c, _L - 1) + c * _L
    for cb in range(d // _L):
        idxtab[2 * _L + cb] = lane + cb * _L

    def idx_start(g, b):
        pltpu.async_copy(idx_hbm.at[pl.ds(g * nb + ibase, _CHUNK)],
                         idx_v.at[b], sem_idx.at[b])

    def gather_start(b):
        pltpu.async_copy(table_hbm.at[idx_v.at[b]], rows_v.at[b],
                         sem_gth.at[b])

    def wait_idx(b):
        pltpu.make_async_copy(idx_hbm.at[pl.ds(0, _CHUNK)], idx_v.at[b],
                              sem_idx.at[b]).wait()

    def wait_gth(b):
        pltpu.make_async_copy(table_hbm.at[pl.ds(0, _CHUNK)], rows_v.at[b],
                              sem_gth.at[b]).wait()

    def wait_out(b):
        for _ in range(nct):
            pltpu.make_async_copy(trans_v.at[b, pl.ds(0, _TPW * _BT * 8)],
                                  out_hbm.at[0, 0, pl.ds(0, _TPW * _BT * 8)],
                                  sem_out.at[b]).wait()

    def transpose(b):
        ncb = d // _L

        def rb_step(rbo, _):
            for rp in range(2):
                rb = rbo * 2 + rp
                r0 = rb * _L
                dbase = (rb // 8) * (_BT * 8) + (rb % 8) * _L
                cols = [idxtab[2 * _L + cb] for cb in range(ncb)]
                for r in range(_L):
                    row = jnp.broadcast_to(r0 + r, (_L,))
                    ska = idxtab[r]
                    for cb in range(ncb):
                        sk = skew_v.at[pl.ds((rp * ncb + cb) * (16 * _L),
                                             16 * _L)]
                        v = plsc.load_gather(rows_v.at[b], [row, cols[cb]])
                        plsc.store_scatter(sk, [ska], v)
                for c in range(_L):
                    skb = idxtab[_L + c]
                    for cb in range(ncb):
                        sk = skew_v.at[pl.ds((rp * ncb + cb) * (16 * _L),
                                             16 * _L)]
                        w = plsc.load_gather(sk, [skb])
                        ct = cb * (_L // 8) + c // 8
                        dst = ct * (_TPW * _BT * 8) + dbase + (c % 8) * _BT
                        trans_v[b, pl.ds(dst, _L)] = w
            return 0

        lax.fori_loop(0, _CHUNK // _L // 2, rb_step, 0)

    def store_start(g, b):
        for ct in range(nct):
            pltpu.async_copy(
                trans_v.at[b, pl.ds(ct * (_TPW * _BT * 8), _TPW * _BT * 8)],
                out_hbm.at[g, ct, pl.ds(obase, _TPW * _BT * 8)],
                sem_out.at[b])

    idx_start(0, 0)
    idx_start(1, 1)
    wait_idx(0)
    gather_start(0)

    def super_step(s, _):
        for k in range(_NBUF):
            g = s * _NBUF + k
            b = k
            nb_ = 1 - k
            wait_gth(b)

            @pl.when(g + 2 < t_len)
            def _():
                idx_start(g + 2, b)

            @pl.when(g + 1 < t_len)
            def _():
                wait_idx(nb_)
                gather_start(nb_)

            @pl.when(s > 0)
            def _():
                wait_out(b)

            transpose(b)
            store_start(g, b)
        return 0

    lax.fori_loop(0, t_len // _NBUF, super_step, 0)
    wait_out(0)
    wait_out(1)


@functools.lru_cache(maxsize=None)
def _make_gather(nb, t_len, d):
    assert nb % (_NW * _TPW * _BT) == 0 or nb == _NW * _TPW * _BT
    assert d % 8 == 0 and t_len % _NBUF == 0
    mesh = plsc.VectorSubcoreMesh(core_axis_name="c", subcore_axis_name="s")
    return pl.kernel(
        functools.partial(_emb_body, nb=nb, t_len=t_len, d=d),
        mesh=mesh,
        out_type=jax.ShapeDtypeStruct((t_len, d // 8, nb * 8), jnp.float32),
        scratch_types=[
            pltpu.VMEM((_NBUF, _CHUNK), jnp.int32),
            pltpu.VMEM((_NBUF, _CHUNK, d), jnp.float32),
            pltpu.VMEM((4 * 16 * _L,), jnp.float32),
            pltpu.VMEM((2 * 16 + 2, _L), jnp.int32),
            pltpu.VMEM((_NBUF, _CHUNK * d), jnp.float32),
            [pltpu.SemaphoreType.DMA((_NBUF,))] * 3,
        ],
        compiler_params=pltpu.CompilerParams(use_tc_tiling_on_sc=False,
                                             needs_layout_passes=False),
    )


def kernel(words_seq, table):
    nb, t_len = words_seq.shape
    d = table.shape[1]
    idx_t = words_seq.T.reshape(-1).astype(jnp.int32)
    y = _make_gather(nb, t_len, d)(table, idx_t)
    y5 = y.reshape(t_len, d // 8, nb // _BT, 8, _BT)
    return y5.transpose(2, 4, 0, 1, 3).reshape(nb, t_len, d)


# speedup vs baseline: 1.1947x; 1.1947x over previous
import functools

import jax
import jax.numpy as jnp
from jax import lax
from jax.experimental import pallas as pl
from jax.experimental.pallas import tpu as pltpu
from jax.experimental.pallas import tpu_sc as plsc

_NC = 2
_NS = 16
_NW = _NC * _NS
_L = 16
_BT = 128
_TPW = 4
_CHUNK = _TPW * _BT
_NBUF = 2


def _emb_body(table_hbm, idx_hbm, out_hbm, idx_v, rows_v, sk0, sk1, sk2,
              sk3, idxtab, trans_v, sems, *, nb, t_len, d):
    skews = (sk0, sk1, sk2, sk3)
    wid = lax.axis_index("s") * _NC + lax.axis_index("c")
    ibase = wid * _CHUNK
    obase = wid * _TPW * _BT * 8
    nct = d // 8
    sem_idx, sem_gth, sem_out = sems
    lane = lax.iota(jnp.int32, _L)
    for r in range(_L):
        idxtab[r] = jnp.bitwise_and(lane + r, _L - 1) + lane * _L
    for c in range(_L):
        idxtab[_L + c] = jnp.bitwise_and(lane + c, _L - 1) + c * _L
    for cb in range(d // _L):
        idxtab[2 * _L + cb] = lane + cb * _L

    def idx_start(g, b):
        pltpu.async_copy(idx_hbm.at[pl.ds(g * nb + ibase, _CHUNK)],
                         idx_v.at[b], sem_idx.at[b])

    def gather_start(b):
        pltpu.async_copy(table_hbm.at[idx_v.at[b]], rows_v.at[b],
                         sem_gth.at[b])

    def wait_idx(b):
        pltpu.make_async_copy(idx_hbm.at[pl.ds(0, _CHUNK)], idx_v.at[b],
                              sem_idx.at[b]).wait()

    def wait_gth(b):
        pltpu.make_async_copy(table_hbm.at[pl.ds(0, _CHUNK)], rows_v.at[b],
                              sem_gth.at[b]).wait()

    def wait_out(b):
        for _ in range(nct):
            pltpu.make_async_copy(trans_v.at[b, pl.ds(0, _TPW * _BT * 8)],
                                  out_hbm.at[0, 0, pl.ds(0, _TPW * _BT * 8)],
                                  sem_out.at[b]).wait()

    def transpose(b):
        ncb = d // _L
        combos = [(rp, cb) for rp in range(2) for cb in range(ncb)]

        def rb_step(rbo, _):
            for r in range(_L):
                ska = idxtab[r]
                for q, (rp, cb) in enumerate(combos):
                    rr = (rbo * 2 + rp) * _L + r
                    v = rows_v[b, rr, pl.ds(cb * _L, _L)]
                    plsc.store_scatter(skews[q], [ska], v)
            for c in range(_L):
                skb = idxtab[_L + c]
                for q, (rp, cb) in enumerate(combos):
                    rb = rbo * 2 + rp
                    dbase = (rb // 8) * (_BT * 8) + (rb % 8) * _L
                    ct = cb * (_L // 8) + c // 8
                    w = plsc.load_gather(skews[q], [skb])
                    dst = ct * (_TPW * _BT * 8) + dbase + (c % 8) * _BT
                    trans_v[b, pl.ds(dst, _L)] = w
            return 0

        lax.fori_loop(0, _CHUNK // _L // 2, rb_step, 0)

    def store_start(g, b):
        for ct in range(nct):
            pltpu.async_copy(
                trans_v.at[b, pl.ds(ct * (_TPW * _BT * 8), _TPW * _BT * 8)],
                out_hbm.at[g, ct, pl.ds(obase, _TPW * _BT * 8)],
                sem_out.at[b])

    idx_start(0, 0)
    idx_start(1, 1)
    wait_idx(0)
    gather_start(0)

    def super_step(s, _):
        for k in range(_NBUF):
            g = s * _NBUF + k
            b = k
            nb_ = 1 - k
            wait_gth(b)

            @pl.when(g + 2 < t_len)
            def _():
                idx_start(g + 2, b)

            @pl.when(g + 1 < t_len)
            def _():
                wait_idx(nb_)
                gather_start(nb_)

            @pl.when(s > 0)
            def _():
                wait_out(b)

            transpose(b)
            store_start(g, b)
        return 0

    lax.fori_loop(0, t_len // _NBUF, super_step, 0)
    wait_out(0)
    wait_out(1)


@functools.lru_cache(maxsize=None)
def _make_gather(nb, t_len, d):
    assert nb % (_NW * _TPW * _BT) == 0 or nb == _NW * _TPW * _BT
    assert d % 8 == 0 and t_len % _NBUF == 0
    mesh = plsc.VectorSubcoreMesh(core_axis_name="c", subcore_axis_name="s")
    return pl.kernel(
        functools.partial(_emb_body, nb=nb, t_len=t_len, d=d),
        mesh=mesh,
        out_type=jax.ShapeDtypeStruct((t_len, d // 8, nb * 8), jnp.float32),
        scratch_types=[
            pltpu.VMEM((_NBUF, _CHUNK), jnp.int32),
            pltpu.VMEM((_NBUF, _CHUNK, d), jnp.float32),
            pltpu.VMEM((16 * _L,), jnp.float32),
            pltpu.VMEM((16 * _L,), jnp.float32),
            pltpu.VMEM((16 * _L,), jnp.float32),
            pltpu.VMEM((16 * _L,), jnp.float32),
            pltpu.VMEM((2 * 16 + 2, _L), jnp.int32),
            pltpu.VMEM((_NBUF, _CHUNK * d), jnp.float32),
            [pltpu.SemaphoreType.DMA((_NBUF,))] * 3,
        ],
        compiler_params=pltpu.CompilerParams(use_tc_tiling_on_sc=False,
                                             needs_layout_passes=False),
    )


def kernel(words_seq, table):
    nb, t_len = words_seq.shape
    d = table.shape[1]
    idx_t = words_seq.T.reshape(-1).astype(jnp.int32)
    y = _make_gather(nb, t_len, d)(table, idx_t)
    y5 = y.reshape(t_len, d // 8, nb // _BT, 8, _BT)
    return y5.transpose(2, 4, 0, 1, 3).reshape(nb, t_len, d)


# speedup vs baseline: 1.3139x; 1.0998x over previous
import functools

import jax
import jax.numpy as jnp
from jax import lax
from jax.experimental import pallas as pl
from jax.experimental.pallas import tpu as pltpu
from jax.experimental.pallas import tpu_sc as plsc

_NC = 2
_NS = 16
_NW = _NC * _NS
_L = 16
_BT = 128
_TPW = 4
_CHUNK = _TPW * _BT
_NBUF = 2


def _emb_body(table_hbm, idx_hbm, out_hbm, idx_v, rows_v, sk0, sk1, sk2,
              sk3, sk4, sk5, sk6, sk7, idxtab, trans_v, sems, *, nb, t_len,
              d):
    skews = (sk0, sk1, sk2, sk3, sk4, sk5, sk6, sk7)
    wid = lax.axis_index("s") * _NC + lax.axis_index("c")
    ibase = wid * _CHUNK
    obase = wid * _TPW * _BT * 8
    nct = d // 8
    sem_idx, sem_gth, sem_out = sems
    lane = lax.iota(jnp.int32, _L)
    for r in range(_L):
        idxtab[r] = jnp.bitwise_and(lane + r, _L - 1) + lane * _L
    for c in range(_L):
        idxtab[_L + c] = jnp.bitwise_and(lane + c, _L - 1) + c * _L
    for cb in range(d // _L):
        idxtab[2 * _L + cb] = lane + cb * _L

    def idx_start(g, b):
        pltpu.async_copy(idx_hbm.at[pl.ds(g * nb + ibase, _CHUNK)],
                         idx_v.at[b], sem_idx.at[b])

    def gather_start(b):
        pltpu.async_copy(table_hbm.at[idx_v.at[b]], rows_v.at[b],
                         sem_gth.at[b])

    def wait_idx(b):
        pltpu.make_async_copy(idx_hbm.at[pl.ds(0, _CHUNK)], idx_v.at[b],
                              sem_idx.at[b]).wait()

    def wait_gth(b):
        pltpu.make_async_copy(table_hbm.at[pl.ds(0, _CHUNK)], rows_v.at[b],
                              sem_gth.at[b]).wait()

    def wait_out(b):
        for _ in range(nct):
            pltpu.make_async_copy(trans_v.at[b, pl.ds(0, _TPW * _BT * 8)],
                                  out_hbm.at[0, 0, pl.ds(0, _TPW * _BT * 8)],
                                  sem_out.at[b]).wait()

    def transpose(b):
        ncb = d // _L
        combos = [(rpp, cb) for rpp in range(4) for cb in range(ncb)]

        def rb_step(rbo, _):
            for r in range(_L):
                ska = idxtab[r]
                for q, (rpp, cb) in enumerate(combos):
                    rr = (rbo * 4 + rpp) * _L + r
                    v = rows_v[b, rr, pl.ds(cb * _L, _L)]
                    plsc.store_scatter(skews[q], [ska], v)
            for c in range(_L):
                skb = idxtab[_L + c]
                for q, (rpp, cb) in enumerate(combos):
                    rb = rbo * 4 + rpp
                    dbase = (rb // 8) * (_BT * 8) + (rb % 8) * _L
                    ct = cb * (_L // 8) + c // 8
                    w = plsc.load_gather(skews[q], [skb])
                    dst = ct * (_TPW * _BT * 8) + dbase + (c % 8) * _BT
                    trans_v[b, pl.ds(dst, _L)] = w
            return 0

        lax.fori_loop(0, _CHUNK // _L // 4, rb_step, 0)

    def store_start(g, b):
        for ct in range(nct):
            pltpu.async_copy(
                trans_v.at[b, pl.ds(ct * (_TPW * _BT * 8), _TPW * _BT * 8)],
                out_hbm.at[g, ct, pl.ds(obase, _TPW * _BT * 8)],
                sem_out.at[b])

    idx_start(0, 0)
    idx_start(1, 1)
    wait_idx(0)
    gather_start(0)

    def super_step(s, _):
        for k in range(_NBUF):
            g = s * _NBUF + k
            b = k
            nb_ = 1 - k
            wait_gth(b)

            @pl.when(g + 2 < t_len)
            def _():
                idx_start(g + 2, b)

            @pl.when(g + 1 < t_len)
            def _():
                wait_idx(nb_)
                gather_start(nb_)

            @pl.when(s > 0)
            def _():
                wait_out(b)

            transpose(b)
            store_start(g, b)
        return 0

    lax.fori_loop(0, t_len // _NBUF, super_step, 0)
    wait_out(0)
    wait_out(1)


@functools.lru_cache(maxsize=None)
def _make_gather(nb, t_len, d):
    assert nb % (_NW * _TPW * _BT) == 0 or nb == _NW * _TPW * _BT
    assert d % 8 == 0 and t_len % _NBUF == 0
    mesh = plsc.VectorSubcoreMesh(core_axis_name="c", subcore_axis_name="s")
    return pl.kernel(
        functools.partial(_emb_body, nb=nb, t_len=t_len, d=d),
        mesh=mesh,
        out_type=jax.ShapeDtypeStruct((t_len, d // 8, nb * 8), jnp.float32),
        scratch_types=[
            pltpu.VMEM((_NBUF, _CHUNK), jnp.int32),
            pltpu.VMEM((_NBUF, _CHUNK, d), jnp.float32),
            pltpu.VMEM((16 * _L,), jnp.float32),
            pltpu.VMEM((16 * _L,), jnp.float32),
            pltpu.VMEM((16 * _L,), jnp.float32),
            pltpu.VMEM((16 * _L,), jnp.float32),
            pltpu.VMEM((16 * _L,), jnp.float32),
            pltpu.VMEM((16 * _L,), jnp.float32),
            pltpu.VMEM((16 * _L,), jnp.float32),
            pltpu.VMEM((16 * _L,), jnp.float32),
            pltpu.VMEM((2 * 16 + 2, _L), jnp.int32),
            pltpu.VMEM((_NBUF, _CHUNK * d), jnp.float32),
            [pltpu.SemaphoreType.DMA((_NBUF,))] * 3,
        ],
        compiler_params=pltpu.CompilerParams(use_tc_tiling_on_sc=False,
                                             needs_layout_passes=False),
    )


def kernel(words_seq, table):
    nb, t_len = words_seq.shape
    d = table.shape[1]
    idx_t = words_seq.T.reshape(-1).astype(jnp.int32)
    y = _make_gather(nb, t_len, d)(table, idx_t)
    y5 = y.reshape(t_len, d // 8, nb // _BT, 8, _BT)
    return y5.transpose(2, 4, 0, 1, 3).reshape(nb, t_len, d)


# speedup vs baseline: 3.3072x; 2.5170x over previous
import functools

import jax
import jax.numpy as jnp
from jax import lax
from jax.experimental import pallas as pl
from jax.experimental.pallas import tpu as pltpu
from jax.experimental.pallas import tpu_sc as plsc

_NC = 2
_NS = 16
_NW = _NC * _NS
_L = 16
_BT = 128
_TPW = 4
_CHUNK = _TPW * _BT
_NBUF = 2


def _emb_body(table_hbm, idx_hbm, out_hbm, idx_v, rows_v, sk0, sk1, sk2,
              sk3, sk4, sk5, sk6, sk7, idxtab, trans_v, sems, *, nb, t_len,
              d):
    skews = (sk0, sk1, sk2, sk3, sk4, sk5, sk6, sk7)
    wid = lax.axis_index("s") * _NC + lax.axis_index("c")
    ibase = wid * _CHUNK
    obase = wid * _TPW * _BT * 8
    nct = d // 8
    sem_idx, sem_gth, sem_out = sems
    lane = lax.iota(jnp.int32, _L)
    for r in range(_L):
        idxtab[r] = jnp.bitwise_and(lane + r, _L - 1) + lane * _L
    for c in range(_L):
        idxtab[_L + c] = jnp.bitwise_and(lane + c, _L - 1) + c * _L
    for cb in range(d // _L):
        idxtab[2 * _L + cb] = lane + cb * _L

    def idx_start(g, b):
        pltpu.async_copy(idx_hbm.at[pl.ds(g * nb + ibase, _CHUNK)],
                         idx_v.at[b], sem_idx.at[b])

    def gather_start(b):
        pltpu.async_copy(table_hbm.at[idx_v.at[b]], rows_v.at[b],
                         sem_gth.at[b])

    def wait_idx(b):
        pltpu.make_async_copy(idx_hbm.at[pl.ds(0, _CHUNK)], idx_v.at[b],
                              sem_idx.at[b]).wait()

    def wait_gth(b):
        pltpu.make_async_copy(table_hbm.at[pl.ds(0, _CHUNK)], rows_v.at[b],
                              sem_gth.at[b]).wait()

    def wait_out(b):
        for _ in range(nct):
            pltpu.make_async_copy(trans_v.at[b, pl.ds(0, _TPW * _BT * 8)],
                                  out_hbm.at[0, 0, pl.ds(0, _TPW * _BT * 8)],
                                  sem_out.at[b]).wait()

    def transpose(b):
        ncb = d // _L
        combos = [(rpp, cb) for rpp in range(4) for cb in range(ncb)]

        def rb_step(rbo, _):
            for r in range(_L):
                ska = idxtab[r]
                vs = []
                for q, (rpp, cb) in enumerate(combos):
                    rr = (rbo * 4 + rpp) * _L + r
                    vs.append(rows_v[b, rr, pl.ds(cb * _L, _L)])
                for q in range(len(combos)):
                    plsc.store_scatter(skews[q], [ska], vs[q])
            for c in range(_L):
                skb = idxtab[_L + c]
                ws = [plsc.load_gather(skews[q], [skb])
                      for q in range(len(combos))]
                for q, (rpp, cb) in enumerate(combos):
                    rb = rbo * 4 + rpp
                    dbase = (rb // 8) * (_BT * 8) + (rb % 8) * _L
                    ct = cb * (_L // 8) + c // 8
                    dst = ct * (_TPW * _BT * 8) + dbase + (c % 8) * _BT
                    trans_v[b, pl.ds(dst, _L)] = ws[q]
            return 0

        lax.fori_loop(0, _CHUNK // _L // 4, rb_step, 0)

    def store_start(g, b):
        for ct in range(nct):
            pltpu.async_copy(
                trans_v.at[b, pl.ds(ct * (_TPW * _BT * 8), _TPW * _BT * 8)],
                out_hbm.at[g, ct, pl.ds(obase, _TPW * _BT * 8)],
                sem_out.at[b])

    idx_start(0, 0)
    idx_start(1, 1)
    wait_idx(0)
    gather_start(0)

    def super_step(s, _):
        for k in range(_NBUF):
            g = s * _NBUF + k
            b = k
            nb_ = 1 - k
            wait_gth(b)

            @pl.when(g + 2 < t_len)
            def _():
                idx_start(g + 2, b)

            @pl.when(g + 1 < t_len)
            def _():
                wait_idx(nb_)
                gather_start(nb_)

            @pl.when(s > 0)
            def _():
                wait_out(b)

            transpose(b)
            store_start(g, b)
        return 0

    lax.fori_loop(0, t_len // _NBUF, super_step, 0)
    wait_out(0)
    wait_out(1)


@functools.lru_cache(maxsize=None)
def _make_gather(nb, t_len, d):
    assert nb % (_NW * _TPW * _BT) == 0 or nb == _NW * _TPW * _BT
    assert d % 8 == 0 and t_len % _NBUF == 0
    mesh = plsc.VectorSubcoreMesh(core_axis_name="c", subcore_axis_name="s")
    return pl.kernel(
        functools.partial(_emb_body, nb=nb, t_len=t_len, d=d),
        mesh=mesh,
        out_type=jax.ShapeDtypeStruct((t_len, d // 8, nb * 8), jnp.float32),
        scratch_types=[
            pltpu.VMEM((_NBUF, _CHUNK), jnp.int32),
            pltpu.VMEM((_NBUF, _CHUNK, d), jnp.float32),
            pltpu.VMEM((16 * _L,), jnp.float32),
            pltpu.VMEM((16 * _L,), jnp.float32),
            pltpu.VMEM((16 * _L,), jnp.float32),
            pltpu.VMEM((16 * _L,), jnp.float32),
            pltpu.VMEM((16 * _L,), jnp.float32),
            pltpu.VMEM((16 * _L,), jnp.float32),
            pltpu.VMEM((16 * _L,), jnp.float32),
            pltpu.VMEM((16 * _L,), jnp.float32),
            pltpu.VMEM((2 * 16 + 2, _L), jnp.int32),
            pltpu.VMEM((_NBUF, _CHUNK * d), jnp.float32),
            [pltpu.SemaphoreType.DMA((_NBUF,))] * 3,
        ],
        compiler_params=pltpu.CompilerParams(use_tc_tiling_on_sc=False,
                                             needs_layout_passes=False),
    )


def kernel(words_seq, table):
    nb, t_len = words_seq.shape
    d = table.shape[1]
    idx_t = words_seq.T.reshape(-1).astype(jnp.int32)
    y = _make_gather(nb, t_len, d)(table, idx_t)
    y5 = y.reshape(t_len, d // 8, nb // _BT, 8, _BT)
    return y5.transpose(2, 4, 0, 1, 3).reshape(nb, t_len, d)
